# Initial kernel scaffold; baseline (speedup 1.0000x reference)
import jax
import jax.numpy as jnp
from jax.experimental import pallas as pl


def kernel(qk, v, random_rotations):
    raise NotImplementedError("write your pallas kernel here")


# speedup vs baseline: 43.9451x; 43.9451x over previous
import functools

import jax
import jax.numpy as jnp
from jax.experimental import pallas as pl

B = 16
S = 2048
D = 64
N_HASHES = 8
N_BUCKETS = 32
S_T = 512


def _lsh_kernel(qk_ref, v_ref, w_ref, out_ref, bkt_ref):
    qk = qk_ref[0]
    rotated = jnp.dot(qk, w_ref[...], preferred_element_type=jnp.float32)
    cols = []
    for h in range(N_HASHES):
        xs = rotated[:, h * 16:(h + 1) * 16]
        m1 = jnp.max(xs, axis=1)
        a1 = jnp.argmax(xs, axis=1).astype(jnp.int32)
        m2 = -jnp.min(xs, axis=1)
        a2 = jnp.argmin(xs, axis=1).astype(jnp.int32)
        b_h = jnp.where(m1 >= m2, a1, a2 + 16) + h * N_BUCKETS
        cols.append(b_h[:, None])
    bkt_ref[0] = jnp.concatenate(cols, axis=1)
    out_ref[0] = v_ref[0]


@jax.jit
def kernel(qk, v, random_rotations):
    w = random_rotations[0].reshape(D, N_HASHES * 16)
    grid = (B, S // S_T)
    out, bkt = pl.pallas_call(
        _lsh_kernel,
        grid=grid,
        in_specs=[
            pl.BlockSpec((1, S_T, D), lambda b, s: (b, s, 0)),
            pl.BlockSpec((1, S_T, D), lambda b, s: (b, s, 0)),
            pl.BlockSpec((D, N_HASHES * 16), lambda b, s: (0, 0)),
        ],
        out_specs=[
            pl.BlockSpec((1, S_T, D), lambda b, s: (b, s, 0)),
            pl.BlockSpec((1, S_T, N_HASHES), lambda b, s: (b, s, 0)),
        ],
        out_shape=[
            jax.ShapeDtypeStruct((B, S, D), jnp.float32),
            jax.ShapeDtypeStruct((B, S, N_HASHES), jnp.int32),
        ],
    )(qk, v, w)
    buckets = bkt.transpose(0, 2, 1).reshape(B, N_HASHES * S)
    return out, buckets


# speedup vs baseline: 116.0101x; 2.6399x over previous
import functools

import jax
import jax.numpy as jnp
from jax.experimental import pallas as pl

B = 16
S = 2048
D = 64
N_HASHES = 8
N_BUCKETS = 32
S_T = 512


def _lsh_kernel(qk_ref, v_ref, wt_ref, out_ref, bkt_ref):
    qk = qk_ref[0]
    rot = jax.lax.dot_general(wt_ref[...], qk, (((1,), (1,)), ((), ())),
                              preferred_element_type=jnp.float32)
    rows = []
    for h in range(N_HASHES):
        xs = rot[h * 16:(h + 1) * 16, :]
        m1 = jnp.max(xs, axis=0)
        a1 = jnp.argmax(xs, axis=0).astype(jnp.int32)
        m2 = -jnp.min(xs, axis=0)
        a2 = jnp.argmin(xs, axis=0).astype(jnp.int32)
        b_h = jnp.where(m1 >= m2, a1, a2 + 16) + h * N_BUCKETS
        rows.append(b_h[None, :])
    bkt_ref[0] = jnp.concatenate(rows, axis=0)
    out_ref[0] = v_ref[0]


@jax.jit
def kernel(qk, v, random_rotations):
    wt = random_rotations[0].reshape(D, N_HASHES * 16).T
    grid = (B, S // S_T)
    out, bkt = pl.pallas_call(
        _lsh_kernel,
        grid=grid,
        in_specs=[
            pl.BlockSpec((1, S_T, D), lambda b, s: (b, s, 0)),
            pl.BlockSpec((1, S_T, D), lambda b, s: (b, s, 0)),
            pl.BlockSpec((N_HASHES * 16, D), lambda b, s: (0, 0)),
        ],
        out_specs=[
            pl.BlockSpec((1, S_T, D), lambda b, s: (b, s, 0)),
            pl.BlockSpec((1, N_HASHES, S_T), lambda b, s: (b, 0, s)),
        ],
        out_shape=[
            jax.ShapeDtypeStruct((B, S, D), jnp.float32),
            jax.ShapeDtypeStruct((B, N_HASHES, S), jnp.int32),
        ],
    )(qk, v, wt)
    buckets = bkt.reshape(B, N_HASHES * S)
    return out, buckets


# speedup vs baseline: 157.6193x; 1.3587x over previous
import functools

import jax
import jax.numpy as jnp
from jax.experimental import pallas as pl

B = 16
S = 2048
D = 64
N_HASHES = 8
N_BUCKETS = 32
S_T = 2048


def _lsh_kernel(qk_ref, v_ref, wt_ref, out_ref, bkt_ref):
    qk = qk_ref[0]
    rot = jax.lax.dot_general(wt_ref[...], qk, (((1,), (1,)), ((), ())),
                              preferred_element_type=jnp.float32)
    rows = []
    for h in range(N_HASHES):
        xs = rot[h * 16:(h + 1) * 16, :]
        m1 = jnp.max(xs, axis=0)
        a1 = jnp.argmax(xs, axis=0).astype(jnp.int32)
        m2 = -jnp.min(xs, axis=0)
        a2 = jnp.argmin(xs, axis=0).astype(jnp.int32)
        b_h = jnp.where(m1 >= m2, a1, a2 + 16) + h * N_BUCKETS
        rows.append(b_h[None, :])
    bkt_ref[0] = jnp.concatenate(rows, axis=0)
    out_ref[0] = v_ref[0]


@jax.jit
def kernel(qk, v, random_rotations):
    wt = random_rotations[0].reshape(D, N_HASHES * 16).T
    grid = (B, S // S_T)
    out, bkt = pl.pallas_call(
        _lsh_kernel,
        grid=grid,
        in_specs=[
            pl.BlockSpec((1, S_T, D), lambda b, s: (b, s, 0)),
            pl.BlockSpec((1, S_T, D), lambda b, s: (b, s, 0)),
            pl.BlockSpec((N_HASHES * 16, D), lambda b, s: (0, 0)),
        ],
        out_specs=[
            pl.BlockSpec((1, S_T, D), lambda b, s: (b, s, 0)),
            pl.BlockSpec((1, N_HASHES, S_T), lambda b, s: (b, 0, s)),
        ],
        out_shape=[
            jax.ShapeDtypeStruct((B, S, D), jnp.float32),
            jax.ShapeDtypeStruct((B, N_HASHES, S), jnp.int32),
        ],
    )(qk, v, wt)
    buckets = bkt.reshape(B, N_HASHES * S)
    return out, buckets


# speedup vs baseline: 239.5699x; 1.5199x over previous
import functools

import jax
import jax.numpy as jnp
from jax.experimental import pallas as pl

B = 16
S = 2048
D = 64
N_HASHES = 8
N_BUCKETS = 32
S_T = 2048


def _lsh_kernel(qk_ref, wt_ref, bkt_ref):
    qk = qk_ref[0]
    rot = jax.lax.dot_general(wt_ref[...], qk, (((1,), (1,)), ((), ())),
                              preferred_element_type=jnp.float32)
    rows = []
    for h in range(N_HASHES):
        xs = rot[h * 16:(h + 1) * 16, :]
        m1 = jnp.max(xs, axis=0)
        a1 = jnp.argmax(xs, axis=0).astype(jnp.int32)
        m2 = -jnp.min(xs, axis=0)
        a2 = jnp.argmin(xs, axis=0).astype(jnp.int32)
        b_h = jnp.where(m1 >= m2, a1, a2 + 16) + h * N_BUCKETS
        rows.append(b_h[None, :])
    bkt_ref[0] = jnp.concatenate(rows, axis=0)


@jax.jit
def kernel(qk, v, random_rotations):
    wt = random_rotations[0].reshape(D, N_HASHES * 16).T
    grid = (B, S // S_T)
    bkt = pl.pallas_call(
        _lsh_kernel,
        grid=grid,
        in_specs=[
            pl.BlockSpec((1, S_T, D), lambda b, s: (b, s, 0)),
            pl.BlockSpec((N_HASHES * 16, D), lambda b, s: (0, 0)),
        ],
        out_specs=pl.BlockSpec((1, N_HASHES, S_T), lambda b, s: (b, 0, s)),
        out_shape=jax.ShapeDtypeStruct((B, N_HASHES, S), jnp.int32),
    )(qk, wt)
    buckets = bkt.reshape(B, N_HASHES * S)
    return v, buckets


# speedup vs baseline: 254.2530x; 1.0613x over previous
import functools

import jax
import jax.numpy as jnp
from jax.experimental import pallas as pl

B = 16
S = 2048
D = 64
N_HASHES = 8
N_BUCKETS = 32
S_T = 2048


def _lsh_kernel(qk_ref, wt_ref, bkt_ref):
    qk = qk_ref[0]
    rot = jax.lax.dot_general(wt_ref[...], qk, (((1,), (1,)), ((), ())),
                              preferred_element_type=jnp.float32)
    x3 = rot.reshape(N_HASHES, 16, S_T)
    m1 = jnp.max(x3, axis=1)
    m2 = jnp.min(x3, axis=1)
    sel = m1 >= -m2
    target = jnp.where(sel, m1, m2)
    off = jnp.where(sel, 0, 16)
    iota = jax.lax.broadcasted_iota(jnp.int32, (N_HASHES, 16, S_T), 1)
    score = jnp.where(x3 == target[:, None, :], iota + off[:, None, :], 255)
    idx = jnp.min(score, axis=1)
    hbase = jax.lax.broadcasted_iota(jnp.int32, (N_HASHES, S_T), 0) * N_BUCKETS
    bkt_ref[0] = idx + hbase


@jax.jit
def kernel(qk, v, random_rotations):
    wt = random_rotations[0].reshape(D, N_HASHES * 16).T
    grid = (B, S // S_T)
    bkt = pl.pallas_call(
        _lsh_kernel,
        grid=grid,
        in_specs=[
            pl.BlockSpec((1, S_T, D), lambda b, s: (b, s, 0)),
            pl.BlockSpec((N_HASHES * 16, D), lambda b, s: (0, 0)),
        ],
        out_specs=pl.BlockSpec((1, N_HASHES, S_T), lambda b, s: (b, 0, s)),
        out_shape=jax.ShapeDtypeStruct((B, N_HASHES, S), jnp.int32),
    )(qk, wt)
    buckets = bkt.reshape(B, N_HASHES * S)
    return v, buckets
